# Initial kernel scaffold; baseline (speedup 1.0000x reference)
import jax
import jax.numpy as jnp
from jax.experimental import pallas as pl


def kernel(x1, x2, x3, y3, W4, g4, b4, W5, g5, b5, Wm1, bm1, g6, b6, Wm2, bm2, g7, b7, Wm3, bm3, g8, b8):
    raise NotImplementedError("write your pallas kernel here")


# speedup vs baseline: 17.7094x; 17.7094x over previous
import functools

import jax
import jax.numpy as jnp
from jax.experimental import pallas as pl

B, N, M, K = 4, 2048, 2048, 20
TN = 512
NT = N // TN
EPS = 1e-5
NEG = -jnp.inf


def _chan_stats(p):
    s = jnp.sum(p, axis=1, keepdims=True)
    s2 = jnp.sum(p * p, axis=1, keepdims=True)
    return jnp.concatenate([s, s2], axis=1)


def _group_mat(groups, chans):
    per = chans // groups
    g = jax.lax.broadcasted_iota(jnp.int32, (groups, chans), 0)
    c = jax.lax.broadcasted_iota(jnp.int32, (groups, chans), 1)
    return (c // per == g).astype(jnp.float32)


def _gn_scale(gstats, cnt, g_col, b_col, groups, chans):
    mean_g = gstats[:, 0:1] / cnt
    var_g = gstats[:, 1:2] / cnt - mean_g * mean_g
    inv_g = 1.0 / jnp.sqrt(var_g + EPS)
    e = _group_mat(groups, chans)
    mean_c = jax.lax.dot_general(e, mean_g, (((0,), (0,)), ((), ())),
                                 preferred_element_type=jnp.float32)
    inv_c = jax.lax.dot_general(e, inv_g, (((0,), (0,)), ((), ())),
                                preferred_element_type=jnp.float32)
    scale = inv_c * g_col
    shift = b_col - mean_c * scale
    return scale, shift


def _attn_kernel(x3_ref, y3_ref, w4_ref, out4_ref, stats_ref):
    b = pl.program_id(0)
    ni = pl.program_id(1)
    x = x3_ref[0]
    y = y3_ref[0]
    xn = x / jnp.maximum(jnp.sqrt(jnp.sum(x * x, axis=0, keepdims=True)), 1e-12)
    yn = y / jnp.maximum(jnp.sqrt(jnp.sum(y * y, axis=0, keepdims=True)), 1e-12)
    simi = jax.lax.dot_general(xn, yn, (((0,), (0,)), ((), ())),
                               preferred_element_type=jnp.float32)
    m1 = jnp.max(simi, axis=1, keepdims=True)
    run = simi
    thr = m1
    for _ in range(K - 1):
        run = jnp.where(run >= thr, NEG, run)
        thr = jnp.max(run, axis=1, keepdims=True)
    w = jnp.where(simi >= thr, jnp.exp(simi - m1), 0.0)
    y_aug = jnp.concatenate([y, jnp.ones((8, M), jnp.float32)], axis=0)
    num_full = jax.lax.dot_general(y_aug, w, (((1,), (1,)), ((), ())),
                                   preferred_element_type=jnp.float32)
    denom = num_full[128:129, :]
    wnf = num_full[:128, :] / denom - x
    attent = jnp.concatenate([x, wnf], axis=0)
    out4 = jax.lax.dot_general(w4_ref[...], attent, (((1,), (0,)), ((), ())),
                               preferred_element_type=jnp.float32)
    out4_ref[0] = out4

    @pl.when(jnp.logical_and(b == 0, ni == 0))
    def _():
        stats_ref[...] = jnp.zeros_like(stats_ref)

    stats_ref[...] += _chan_stats(out4)


def _conv5_kernel(out4_ref, bnstats_ref, x1_ref, x2_ref, x3_ref,
                  g4_ref, b4_ref, w5_ref, out5_ref, gn5_ref):
    ni = pl.program_id(1)
    cnt = jnp.float32(B * N)
    mean = bnstats_ref[:, 0:1] / cnt
    var = bnstats_ref[:, 1:2] / cnt - mean * mean
    inv = 1.0 / jnp.sqrt(var + EPS)
    x4 = (out4_ref[0] - mean) * inv * g4_ref[...] + b4_ref[...]
    x4 = jnp.where(x4 >= 0, x4, 0.2 * x4)
    xc = jnp.concatenate([x1_ref[0], x2_ref[0], x3_ref[0], x4], axis=0)
    out5 = jax.lax.dot_general(w5_ref[...], xc, (((1,), (0,)), ((), ())),
                               preferred_element_type=jnp.float32)
    out5_ref[0] = out5

    @pl.when(ni == 0)
    def _():
        gn5_ref[...] = jnp.zeros_like(gn5_ref)

    gn5_ref[0] += jax.lax.dot_general(_group_mat(32, 512), _chan_stats(out5),
                                      (((1,), (0,)), ((), ())),
                                      preferred_element_type=jnp.float32)


def _mlp1_kernel(out5_ref, gn5_ref, g5_ref, b5_ref, wm1_ref, bm1_ref,
                 m1_ref, gn6_ref, pmax_ref, psum_ref):
    ni = pl.program_id(1)
    cnt = jnp.float32(16 * N)
    scale, shift = _gn_scale(gn5_ref[0], cnt, g5_ref[...], b5_ref[...], 32, 512)
    lc = out5_ref[0] * scale + shift
    lc = jnp.where(lc >= 0, lc, 0.2 * lc)

    @pl.when(ni == 0)
    def _():
        pmax_ref[...] = jnp.full_like(pmax_ref, NEG)
        psum_ref[...] = jnp.zeros_like(psum_ref)
        gn6_ref[...] = jnp.zeros_like(gn6_ref)

    pmax_ref[0] = jnp.maximum(pmax_ref[0], jnp.max(lc, axis=1, keepdims=True))
    psum_ref[0] += jnp.sum(lc, axis=1, keepdims=True)

    @pl.when(ni == NT - 1)
    def _():
        psum_ref[0] = psum_ref[0] / jnp.float32(N)

    m1 = jax.lax.dot_general(wm1_ref[...], lc, (((1,), (0,)), ((), ())),
                             preferred_element_type=jnp.float32) + bm1_ref[...]
    m1_ref[0] = m1
    gn6_ref[0] += jax.lax.dot_general(_group_mat(32, 512), _chan_stats(m1),
                                      (((1,), (0,)), ((), ())),
                                      preferred_element_type=jnp.float32)


def _mlp_mid_kernel(cin, cout, in_ref, gnin_ref, g_ref, b_ref,
                    w_ref, bias_ref, out_ref, gnout_ref):
    ni = pl.program_id(1)
    cnt = jnp.float32((cin // 32) * N)
    scale, shift = _gn_scale(gnin_ref[0], cnt, g_ref[...], b_ref[...], 32, cin)
    h = in_ref[0] * scale + shift
    h = jnp.maximum(h, 0.0)
    out = jax.lax.dot_general(w_ref[...], h, (((1,), (0,)), ((), ())),
                              preferred_element_type=jnp.float32) + bias_ref[...]
    out_ref[0] = out

    @pl.when(ni == 0)
    def _():
        gnout_ref[...] = jnp.zeros_like(gnout_ref)

    gnout_ref[0] += jax.lax.dot_general(_group_mat(32, cout), _chan_stats(out),
                                        (((1,), (0,)), ((), ())),
                                        preferred_element_type=jnp.float32)


def _final_kernel(in_ref, gnin_ref, g_ref, b_ref, out_ref):
    cnt = jnp.float32(4 * N)
    scale, shift = _gn_scale(gnin_ref[0], cnt, g_ref[...], b_ref[...], 32, 128)
    h = in_ref[0] * scale + shift
    out_ref[0] = jnp.maximum(h, 0.0)


def _col(v):
    return v.reshape(-1, 1)


def kernel(x1, x2, x3, y3, W4, g4, b4, W5, g5, b5, Wm1, bm1, g6, b6,
           Wm2, bm2, g7, b7, Wm3, bm3, g8, b8):
    f32 = jnp.float32
    grid = (B, NT)

    def bspec(c):
        return pl.BlockSpec((1, c, TN), lambda b_, n_: (b_, 0, n_))

    def full(shape):
        return pl.BlockSpec(shape, lambda b_, n_: tuple(0 for _ in shape))

    def perb(shape):
        return pl.BlockSpec((1,) + shape[1:], lambda b_, n_: (b_,) + tuple(0 for _ in shape[1:]))

    out4, bnstats = pl.pallas_call(
        _attn_kernel,
        grid=grid,
        in_specs=[bspec(128),
                  pl.BlockSpec((1, 128, M), lambda b_, n_: (b_, 0, 0)),
                  full((256, 256))],
        out_specs=[bspec(256), full((256, 2))],
        out_shape=[jax.ShapeDtypeStruct((B, 256, N), f32),
                   jax.ShapeDtypeStruct((256, 2), f32)],
    )(x3, y3, W4)

    out5, gn5 = pl.pallas_call(
        _conv5_kernel,
        grid=grid,
        in_specs=[bspec(256), full((256, 2)), bspec(64), bspec(64), bspec(128),
                  full((256, 1)), full((256, 1)), full((512, 512))],
        out_specs=[bspec(512), perb((1, 32, 2))],
        out_shape=[jax.ShapeDtypeStruct((B, 512, N), f32),
                   jax.ShapeDtypeStruct((B, 32, 2), f32)],
    )(out4, bnstats, x1, x2, x3, _col(g4), _col(b4), W5)

    m1p, gn6, pmax, psum = pl.pallas_call(
        _mlp1_kernel,
        grid=grid,
        in_specs=[bspec(512), perb((1, 32, 2)), full((512, 1)), full((512, 1)),
                  full((512, 512)), full((512, 1))],
        out_specs=[bspec(512), perb((1, 32, 2)),
                   perb((1, 512, 1)), perb((1, 512, 1))],
        out_shape=[jax.ShapeDtypeStruct((B, 512, N), f32),
                   jax.ShapeDtypeStruct((B, 32, 2), f32),
                   jax.ShapeDtypeStruct((B, 512, 1), f32),
                   jax.ShapeDtypeStruct((B, 512, 1), f32)],
    )(out5, gn5, _col(g5), _col(b5), Wm1, _col(bm1))

    m2p, gn7 = pl.pallas_call(
        functools.partial(_mlp_mid_kernel, 512, 256),
        grid=grid,
        in_specs=[bspec(512), perb((1, 32, 2)), full((512, 1)), full((512, 1)),
                  full((256, 512)), full((256, 1))],
        out_specs=[bspec(256), perb((1, 32, 2))],
        out_shape=[jax.ShapeDtypeStruct((B, 256, N), f32),
                   jax.ShapeDtypeStruct((B, 32, 2), f32)],
    )(m1p, gn6, _col(g6), _col(b6), Wm2, _col(bm2))

    m3p, gn8 = pl.pallas_call(
        functools.partial(_mlp_mid_kernel, 256, 128),
        grid=grid,
        in_specs=[bspec(256), perb((1, 32, 2)), full((256, 1)), full((256, 1)),
                  full((128, 256)), full((128, 1))],
        out_specs=[bspec(128), perb((1, 32, 2))],
        out_shape=[jax.ShapeDtypeStruct((B, 128, N), f32),
                   jax.ShapeDtypeStruct((B, 32, 2), f32)],
    )(m2p, gn7, _col(g7), _col(b7), Wm3, _col(bm3))

    emb = pl.pallas_call(
        _final_kernel,
        grid=grid,
        in_specs=[bspec(128), perb((1, 32, 2)), full((128, 1)), full((128, 1))],
        out_specs=bspec(128),
        out_shape=jax.ShapeDtypeStruct((B, 128, N), f32),
    )(m3p, gn8, _col(g8), _col(b8))

    global_vector = jnp.concatenate([pmax[:, :, 0], psum[:, :, 0]], axis=1)
    return (emb, global_vector[:, :, None])


# speedup vs baseline: 30.9925x; 1.7501x over previous
import jax
import jax.numpy as jnp
from jax.experimental import pallas as pl

B, N, M, K = 4, 2048, 2048, 20
TN = 512
NT = N // TN
EPS = 1e-5
NEG = -jnp.inf


def _group_mat(groups, chans):
    per = chans // groups
    g = jax.lax.broadcasted_iota(jnp.int32, (groups, chans), 0)
    c = jax.lax.broadcasted_iota(jnp.int32, (groups, chans), 1)
    return (c // per == g).astype(jnp.float32)


def _dot(a, b):
    return jax.lax.dot_general(a, b, (((1,), (0,)), ((), ())),
                               preferred_element_type=jnp.float32)


def _gn_apply(x, groups, g_col, b_col):
    chans = x.shape[0]
    cnt = jnp.float32((chans // groups) * x.shape[1])
    e = _group_mat(groups, chans)
    s = jnp.sum(x, axis=1, keepdims=True)
    s2 = jnp.sum(x * x, axis=1, keepdims=True)
    gs = _dot(e, jnp.concatenate([s, s2], axis=1))
    mean_g = gs[:, 0:1] / cnt
    var_g = gs[:, 1:2] / cnt - mean_g * mean_g
    inv_g = 1.0 / jnp.sqrt(var_g + EPS)
    per = chans // groups
    ci = jax.lax.broadcasted_iota(jnp.int32, (chans, groups), 0)
    gi = jax.lax.broadcasted_iota(jnp.int32, (chans, groups), 1)
    et = (ci // per == gi).astype(jnp.float32)
    mean_c = _dot(et, mean_g)
    inv_c = _dot(et, inv_g)
    scale = inv_c * g_col
    shift = b_col - mean_c * scale
    return x * scale + shift


def _attn_kernel(x3_ref, y3a_ref, w4_ref, out4_ref, stats_ref):
    b = pl.program_id(0)
    ni = pl.program_id(1)
    x = x3_ref[0]
    y_aug = y3a_ref[0]
    y = y_aug[:128]
    xn = x / jnp.maximum(jnp.sqrt(jnp.sum(x * x, axis=0, keepdims=True)), 1e-12)
    yn = y / jnp.maximum(jnp.sqrt(jnp.sum(y * y, axis=0, keepdims=True)), 1e-12)
    simi = jax.lax.dot_general(xn, yn, (((0,), (0,)), ((), ())),
                               preferred_element_type=jnp.float32)
    s0 = 8
    wdt = M // s0
    slices = [simi[:, i * wdt:(i + 1) * wdt] for i in range(s0)]
    c1 = slices[0]
    for s in slices[1:]:
        c1 = jnp.maximum(c1, s)
    c2 = jnp.full_like(c1, NEG)
    for s in slices:
        c2 = jnp.maximum(c2, jnp.where(s >= c1, NEG, s))
    cand = jnp.concatenate([c1, c2], axis=1)
    m1 = jnp.max(c1, axis=1, keepdims=True)
    thr = m1
    for _ in range(K - 1):
        cand = jnp.where(cand >= thr, NEG, cand)
        thr = jnp.max(cand, axis=1, keepdims=True)
    cnt = jnp.sum(jnp.where(simi >= thr, 1.0, 0.0), axis=1, keepdims=True)

    def _next_above(v):
        bits = jax.lax.bitcast_convert_type(v, jnp.int32)
        nb = jnp.where(v >= 0, bits + 1, bits - 1)
        r = jax.lax.bitcast_convert_type(nb, jnp.float32)
        return jnp.where(v == 0.0, jnp.float32(1e-45), r)

    def _cond(state):
        _, c = state
        return jnp.any(c > K)

    def _body(state):
        t, c = state
        masked = jnp.where(simi >= t, simi, jnp.inf)
        mn = jnp.min(masked, axis=1, keepdims=True)
        eqc = jnp.sum(jnp.where(simi == mn, 1.0, 0.0), axis=1, keepdims=True)
        can_drop = jnp.logical_and(c > K, c - eqc >= K)
        new_t = jnp.where(can_drop, _next_above(mn), t)
        new_c = jnp.where(can_drop, c - eqc, jnp.minimum(c, jnp.float32(K)))
        return new_t, new_c

    thr, cnt = jax.lax.while_loop(_cond, _body, (thr, cnt))
    w = jnp.where(simi >= thr, jnp.exp(simi - m1), 0.0)
    num_full = jax.lax.dot_general(y_aug, w, (((1,), (1,)), ((), ())),
                                   preferred_element_type=jnp.float32)
    denom = num_full[128:129, :]
    wnf = num_full[:128, :] / denom - x
    attent = jnp.concatenate([x, wnf], axis=0)
    out4 = _dot(w4_ref[...], attent)
    out4_ref[0] = out4

    @pl.when(jnp.logical_and(b == 0, ni == 0))
    def _():
        stats_ref[...] = jnp.zeros_like(stats_ref)

    s = jnp.sum(out4, axis=1, keepdims=True)
    s2 = jnp.sum(out4 * out4, axis=1, keepdims=True)
    stats_ref[...] += jnp.concatenate([s, s2], axis=1)


def _tail_kernel(out4_ref, bnstats_ref, x1_ref, x2_ref, x3_ref,
                 g4_ref, b4_ref, w5_ref, g5_ref, b5_ref,
                 wm1_ref, bm1_ref, g6_ref, b6_ref,
                 wm2_ref, bm2_ref, g7_ref, b7_ref,
                 wm3_ref, bm3_ref, g8_ref, b8_ref,
                 emb_ref, pmax_ref, pavg_ref):
    cntb = jnp.float32(B * N)
    mean = bnstats_ref[:, 0:1] / cntb
    var = bnstats_ref[:, 1:2] / cntb - mean * mean
    inv = 1.0 / jnp.sqrt(var + EPS)
    x4 = (out4_ref[0] - mean) * inv * g4_ref[...] + b4_ref[...]
    x4 = jnp.where(x4 >= 0, x4, 0.2 * x4)
    xc = jnp.concatenate([x1_ref[0], x2_ref[0], x3_ref[0], x4], axis=0)
    out5 = _dot(w5_ref[...], xc)
    lc = _gn_apply(out5, 32, g5_ref[...], b5_ref[...])
    lc = jnp.where(lc >= 0, lc, 0.2 * lc)
    pmax_ref[0] = jnp.max(lc, axis=1, keepdims=True)
    pavg_ref[0] = jnp.sum(lc, axis=1, keepdims=True) / jnp.float32(N)
    h = _dot(wm1_ref[...], lc) + bm1_ref[...]
    h = jnp.maximum(_gn_apply(h, 32, g6_ref[...], b6_ref[...]), 0.0)
    h = _dot(wm2_ref[...], h) + bm2_ref[...]
    h = jnp.maximum(_gn_apply(h, 32, g7_ref[...], b7_ref[...]), 0.0)
    h = _dot(wm3_ref[...], h) + bm3_ref[...]
    h = jnp.maximum(_gn_apply(h, 32, g8_ref[...], b8_ref[...]), 0.0)
    emb_ref[0] = h


def _col(v):
    return v.reshape(-1, 1)


def kernel(x1, x2, x3, y3, W4, g4, b4, W5, g5, b5, Wm1, bm1, g6, b6,
           Wm2, bm2, g7, b7, Wm3, bm3, g8, b8):
    f32 = jnp.float32

    def bspec(c):
        return pl.BlockSpec((1, c, TN), lambda b_, n_: (b_, 0, n_))

    def full(shape):
        return pl.BlockSpec(shape, lambda b_, n_: tuple(0 for _ in shape))

    y3a = jnp.concatenate(
        [y3, jnp.ones((B, 1, M), f32), jnp.zeros((B, 7, M), f32)], axis=1)
    out4, bnstats = pl.pallas_call(
        _attn_kernel,
        grid=(B, NT),
        in_specs=[bspec(128),
                  pl.BlockSpec((1, 136, M), lambda b_, n_: (b_, 0, 0)),
                  full((256, 256))],
        out_specs=[bspec(256), full((256, 2))],
        out_shape=[jax.ShapeDtypeStruct((B, 256, N), f32),
                   jax.ShapeDtypeStruct((256, 2), f32)],
    )(x3, y3a, W4)

    def fullb(shape):
        return pl.BlockSpec(shape, lambda b_: tuple(0 for _ in shape))

    def bn(c):
        return pl.BlockSpec((1, c, N), lambda b_: (b_, 0, 0))

    emb, pmax, pavg = pl.pallas_call(
        _tail_kernel,
        grid=(B,),
        in_specs=[bn(256), fullb((256, 2)), bn(64), bn(64), bn(128),
                  fullb((256, 1)), fullb((256, 1)), fullb((512, 512)),
                  fullb((512, 1)), fullb((512, 1)),
                  fullb((512, 512)), fullb((512, 1)),
                  fullb((512, 1)), fullb((512, 1)),
                  fullb((256, 512)), fullb((256, 1)),
                  fullb((256, 1)), fullb((256, 1)),
                  fullb((128, 256)), fullb((128, 1)),
                  fullb((128, 1)), fullb((128, 1))],
        out_specs=[bn(128),
                   pl.BlockSpec((1, 512, 1), lambda b_: (b_, 0, 0)),
                   pl.BlockSpec((1, 512, 1), lambda b_: (b_, 0, 0))],
        out_shape=[jax.ShapeDtypeStruct((B, 128, N), f32),
                   jax.ShapeDtypeStruct((B, 512, 1), f32),
                   jax.ShapeDtypeStruct((B, 512, 1), f32)],
    )(out4, bnstats, x1, x2, x3, _col(g4), _col(b4), W5, _col(g5), _col(b5),
      Wm1, _col(bm1), _col(g6), _col(b6), Wm2, _col(bm2), _col(g7), _col(b7),
      Wm3, _col(bm3), _col(g8), _col(b8))

    global_vector = jnp.concatenate([pmax[:, :, 0], pavg[:, :, 0]], axis=1)
    return (emb, global_vector[:, :, None])
